# Initial kernel scaffold; baseline (speedup 1.0000x reference)
import jax
import jax.numpy as jnp
from jax.experimental import pallas as pl


def kernel(input_, weight, weight_table, bias, bias_table):
    raise NotImplementedError("write your pallas kernel here")


# speedup vs baseline: 661.1471x; 661.1471x over previous
import jax
import jax.numpy as jnp
from jax.experimental import pallas as pl
from jax.experimental.pallas import tpu as pltpu

_K = 2048
_N = 2048
_NJ = 256


def _lut(table, idx):
    r = idx.shape[0]
    t_lo = jnp.broadcast_to(table[:, :128], (r, 128))
    t_hi = jnp.broadcast_to(table[:, 128:], (r, 128))
    low = idx & 127
    lo = jnp.take_along_axis(t_lo, low, axis=1, mode="promise_in_bounds")
    hi = jnp.take_along_axis(t_hi, low, axis=1, mode="promise_in_bounds")
    return jnp.where(idx < 128, lo, hi)


def _fused(x_ref, idx_ref, wt_ref, bidx_ref, bt_ref, out_ref):
    idx = idx_ref[...]
    w = _lut(wt_ref[...], idx)
    wb = w.astype(jnp.bfloat16)
    acc = jax.lax.dot_general(
        x_ref[...], wb, (((1,), (1,)), ((), ())),
        preferred_element_type=jnp.float32)
    bidx8 = jnp.broadcast_to(bidx_ref[0], (8, _NJ))
    bvec = _lut(bt_ref[...], bidx8)
    out_ref[...] = acc + bvec[0:1, :]


def kernel(input_, weight, weight_table, bias, bias_table):
    B, M0, K = input_.shape
    M = B * M0
    x = input_.reshape(M, K).astype(jnp.bfloat16)
    wt = weight_table.reshape(1, 256)
    bt = bias_table.reshape(1, 256)
    J = _N // _NJ
    bidx = bias.reshape(J, 1, _NJ)
    out = pl.pallas_call(
        _fused,
        grid=(J,),
        in_specs=[
            pl.BlockSpec((M, _K), lambda j: (0, 0)),
            pl.BlockSpec((_NJ, _K), lambda j: (j, 0)),
            pl.BlockSpec((1, 256), lambda j: (0, 0)),
            pl.BlockSpec((1, 1, _NJ), lambda j: (j, 0, 0)),
            pl.BlockSpec((1, 256), lambda j: (0, 0)),
        ],
        out_specs=pl.BlockSpec((M, _NJ), lambda j: (0, j)),
        out_shape=jax.ShapeDtypeStruct((M, _N), jnp.float32),
    )(x, weight, wt, bidx, bt)
    return out.reshape(B, M0, _N)
